# Initial kernel scaffold; baseline (speedup 1.0000x reference)
import jax
import jax.numpy as jnp
from jax.experimental import pallas as pl


def kernel(x, edge_index, edge_weight, W, b, gamma, beta):
    raise NotImplementedError("write your pallas kernel here")


# speedup vs baseline: 38.2703x; 38.2703x over previous
import functools

import jax
import jax.numpy as jnp
from jax import lax
from jax.experimental import pallas as pl
from jax.experimental.pallas import tpu as pltpu
from jax.experimental.pallas import tpu_sc as plsc

N = 50000
NPAD = 50176
SL = NPAD // 16
E = 800000
LG = 128
NC = 2
NS = 16
NW = NC * NS
GPW = 200
GROUPS = NW * GPW
EPAD = GROUPS * LG
CHG = 40
NCHUNK = GPW // CHG

_f32 = jnp.float32
_i32 = jnp.int32


def _zero_buf(buf):
    def body(i, _):
        buf[pl.ds(i * 16, 16)] = jnp.zeros((16,), _f32)
        return _
    lax.fori_loop(0, SL // 16, body, None)


def _mesh():
    return plsc.VectorSubcoreMesh(core_axis_name="c", subcore_axis_name="s")


def _deg_call(srcg, ewg):
    @functools.partial(
        pl.kernel,
        out_type=jax.ShapeDtypeStruct((NC * NPAD,), _f32),
        mesh=_mesh(),
        scratch_types=[
            pltpu.VMEM((GPW, LG), _i32),
            pltpu.VMEM((GPW, LG), _f32),
            pltpu.VMEM((SL,), _f32),
            pltpu.VMEM_SHARED((NPAD,), _f32),
        ],
    )
    def k(src_ref, ew_ref, out_ref, src_v, ew_v, zbuf_v, acc_sh):
        cid = lax.axis_index("c")
        sid = lax.axis_index("s")
        wid = sid * NC + cid
        _zero_buf(zbuf_v)
        pltpu.sync_copy(zbuf_v, acc_sh.at[pl.ds(sid * SL, SL)])
        plsc.subcore_barrier()
        g0 = wid * GPW
        pltpu.sync_copy(src_ref.at[pl.ds(g0, GPW)], src_v)
        pltpu.sync_copy(ew_ref.at[pl.ds(g0, GPW)], ew_v)

        def grp(j, _):
            pltpu.sync_copy(ew_v.at[j], acc_sh.at[src_v.at[j]], add=True)
            return _
        lax.fori_loop(0, GPW, grp, None)
        plsc.subcore_barrier()
        pltpu.sync_copy(acc_sh.at[pl.ds(sid * SL, SL)], zbuf_v)
        pltpu.sync_copy(zbuf_v, out_ref.at[pl.ds(cid * NPAD + sid * SL, SL)])

    return k(srcg, ewg)


def _dinv_call(degp):
    def body(deg_ref, out_ref):
        d = deg_ref[0:1, :] + deg_ref[1:2, :]
        out_ref[...] = jnp.where(d > 0.0, lax.rsqrt(d), 0.0)

    return pl.pallas_call(
        body,
        out_shape=jax.ShapeDtypeStruct((1, NPAD), _f32),
    )(degp)


def _hop_kernel(stage_tables):
    def k(src_ref, dst_ref, ew_ref, dinv_ref, tab_ref, s_out,
          src_v, dst_v, ew_v, dbuf_v, zbuf_v, bbuf_v, gbuf_v, vbuf_v,
          t0_sh, t1_sh, t2_sh, a0_sh, a1_sh, a2_sh, sem):
        tsh = (t0_sh, t1_sh, t2_sh)
        acc = (a0_sh, a1_sh, a2_sh)
        cid = lax.axis_index("c")
        sid = lax.axis_index("s")
        wid = sid * NC + cid
        sl = pl.ds(sid * SL, SL)
        pltpu.sync_copy(dinv_ref.at[sl], dbuf_v)
        stage_tables(tab_ref, dbuf_v, zbuf_v, bbuf_v, tsh, sl, sid)
        _zero_buf(zbuf_v)
        for c in range(3):
            pltpu.sync_copy(zbuf_v, acc[c].at[sl])
        plsc.subcore_barrier()

        for t in range(NCHUNK):
            base = wid * GPW + t * CHG
            pltpu.sync_copy(src_ref.at[pl.ds(base, CHG)], src_v)
            pltpu.sync_copy(dst_ref.at[pl.ds(base, CHG)], dst_v)
            pltpu.sync_copy(ew_ref.at[pl.ds(base, CHG)], ew_v)

            def grp(j, _):
                for c in range(3):
                    pltpu.async_copy(tsh[c].at[src_v.at[j]], gbuf_v, sem).wait()
                    for i in range(LG // 16):
                        sj = pl.ds(i * 16, 16)
                        vbuf_v[sj] = gbuf_v[sj] * ew_v[j, sj]
                    pltpu.sync_copy(vbuf_v, acc[c].at[dst_v.at[j]], add=True)
                return _
            lax.fori_loop(0, CHG, grp, None)

        plsc.subcore_barrier()
        for c in range(3):
            pltpu.sync_copy(acc[c].at[sl], zbuf_v)
            pltpu.sync_copy(zbuf_v,
                            s_out.at[pl.ds((cid * 3 + c) * NPAD + sid * SL, SL)])

    return k


def _hop_scratch():
    return [
        pltpu.VMEM((CHG, LG), _i32),
        pltpu.VMEM((CHG, LG), _i32),
        pltpu.VMEM((CHG, LG), _f32),
        pltpu.VMEM((SL,), _f32),
        pltpu.VMEM((SL,), _f32),
        pltpu.VMEM((SL,), _f32),
        pltpu.VMEM((LG,), _f32),
        pltpu.VMEM((LG,), _f32),
        pltpu.VMEM_SHARED((NPAD,), _f32),
        pltpu.VMEM_SHARED((NPAD,), _f32),
        pltpu.VMEM_SHARED((NPAD,), _f32),
        pltpu.VMEM_SHARED((NPAD,), _f32),
        pltpu.VMEM_SHARED((NPAD,), _f32),
        pltpu.VMEM_SHARED((NPAD,), _f32),
        pltpu.SemaphoreType.DMA,
    ]


def _hop1_call(srcg, dstg, ewg, dinv, xpad):
    def stage(tab_ref, dbuf_v, zbuf_v, bbuf_v, tsh, sl, sid):
        for c in range(3):
            pltpu.sync_copy(tab_ref.at[pl.ds(c * NPAD + sid * SL, SL)], zbuf_v)

            def mull(i, _):
                sj = pl.ds(i * 16, 16)
                zbuf_v[sj] = zbuf_v[sj] * dbuf_v[sj]
                return _
            lax.fori_loop(0, SL // 16, mull, None)
            pltpu.sync_copy(zbuf_v, tsh[c].at[sl])

    @functools.partial(
        pl.kernel,
        out_type=jax.ShapeDtypeStruct((NC * 3 * NPAD,), _f32),
        mesh=_mesh(),
        scratch_types=_hop_scratch(),
    )
    def k(*args):
        _hop_kernel(stage)(*args)

    return k(srcg, dstg, ewg, dinv, xpad)


def _hop2_call(srcg, dstg, ewg, dinv, s1p):
    def stage(tab_ref, dbuf_v, zbuf_v, bbuf_v, tsh, sl, sid):
        for c in range(3):
            pltpu.sync_copy(tab_ref.at[pl.ds(c * NPAD + sid * SL, SL)], zbuf_v)
            pltpu.sync_copy(tab_ref.at[pl.ds((3 + c) * NPAD + sid * SL, SL)],
                            bbuf_v)

            def mull(i, _):
                sj = pl.ds(i * 16, 16)
                d = dbuf_v[sj]
                zbuf_v[sj] = -(zbuf_v[sj] + bbuf_v[sj]) * d * d
                return _
            lax.fori_loop(0, SL // 16, mull, None)
            pltpu.sync_copy(zbuf_v, tsh[c].at[sl])

    @functools.partial(
        pl.kernel,
        out_type=jax.ShapeDtypeStruct((NC * 3 * NPAD,), _f32),
        mesh=_mesh(),
        scratch_types=_hop_scratch(),
    )
    def k(*args):
        _hop_kernel(stage)(*args)

    return k(srcg, dstg, ewg, dinv, s1p)


_NB = 8
_BL = NPAD // _NB


def _final_call(xpad, s1p, s2p, dinv2, W9, gamma_c, beta_c):
    def body(x_ref, s1_ref, s2_ref, d_ref, w_ref, g_ref, b_ref, out_ref,
             acc_ref, st_ref):
        p = pl.program_id(0)
        j = pl.program_id(1)

        @pl.when(jnp.logical_and(p == 0, j == 0))
        def _():
            acc_ref[...] = jnp.zeros_like(acc_ref)

        nd = -d_ref[...]
        t0 = x_ref[...]
        t1 = nd * (s1_ref[0:3, :] + s1_ref[3:6, :])
        t2 = 2.0 * nd * (s2_ref[0:3, :] + s2_ref[3:6, :]) - t0
        t9 = jnp.concatenate([t0, t1, t2], axis=0)
        out64 = lax.dot_general(w_ref[...], t9, (((0,), (0,)), ((), ())),
                                preferred_element_type=_f32)

        @pl.when(p == 0)
        def _():
            acc_ref[:, 0:1] += jnp.sum(out64, axis=1, keepdims=True)
            acc_ref[:, 1:2] += jnp.sum(out64 * out64, axis=1, keepdims=True)
            out_ref[...] = out64

        @pl.when(p == 1)
        def _():
            @pl.when(j == 0)
            def _():
                mean = acc_ref[:, 0:1] * (1.0 / N)
                var = acc_ref[:, 1:2] * (1.0 / N) - mean * mean
                st_ref[:, 0:1] = mean
                st_ref[:, 1:2] = lax.rsqrt(var + 1e-5)
            mean = st_ref[:, 0:1]
            rstd = st_ref[:, 1:2]
            z = (out64 - mean) * rstd * g_ref[...] + b_ref[...]
            out_ref[...] = jnp.maximum(z, 0.0)

    return pl.pallas_call(
        body,
        grid=(2, _NB),
        in_specs=[
            pl.BlockSpec((3, _BL), lambda p, j: (0, j)),
            pl.BlockSpec((6, _BL), lambda p, j: (0, j)),
            pl.BlockSpec((6, _BL), lambda p, j: (0, j)),
            pl.BlockSpec((1, _BL), lambda p, j: (0, j)),
            pl.BlockSpec((9, 64), lambda p, j: (0, 0)),
            pl.BlockSpec((64, 1), lambda p, j: (0, 0)),
            pl.BlockSpec((64, 1), lambda p, j: (0, 0)),
        ],
        out_specs=pl.BlockSpec((64, _BL), lambda p, j: (0, j)),
        out_shape=jax.ShapeDtypeStruct((64, NPAD), _f32),
        scratch_shapes=[
            pltpu.VMEM((64, 2), _f32),
            pltpu.VMEM((64, 2), _f32),
        ],
    )(xpad, s1p, s2p, dinv2, W9, gamma_c, beta_c)


def kernel(x, edge_index, edge_weight, W, b, gamma, beta):
    del b
    xpad = jnp.pad(x[0], ((0, 0), (0, NPAD - N)))
    npad_e = EPAD - E
    pad_idx = (N + (jnp.arange(npad_e, dtype=_i32) % (NPAD - N)))
    src = jnp.concatenate([edge_index[0], pad_idx]).reshape(GROUPS, LG)
    dst = jnp.concatenate([edge_index[1], pad_idx]).reshape(GROUPS, LG)
    ew = jnp.concatenate([edge_weight,
                          jnp.zeros((npad_e,), _f32)]).reshape(GROUPS, LG)

    degp = _deg_call(src, ew).reshape(NC, NPAD)
    dinv2 = _dinv_call(degp)
    dinv = dinv2.reshape(NPAD)
    s1p = _hop1_call(src, dst, ew, dinv, xpad.reshape(-1))
    s2p = _hop2_call(src, dst, ew, dinv, s1p)

    W9 = W.reshape(9, 64)
    outT = _final_call(xpad, s1p.reshape(6, NPAD), s2p.reshape(6, NPAD),
                       dinv2, W9, gamma.reshape(64, 1), beta.reshape(64, 1))
    return outT[:, :N].reshape(1, 64, N)


# speedup vs baseline: 78.1045x; 2.0409x over previous
import functools

import jax
import jax.numpy as jnp
from jax import lax
from jax.experimental import pallas as pl
from jax.experimental.pallas import tpu as pltpu
from jax.experimental.pallas import tpu_sc as plsc

N = 50000
NPAD = 50176
SL = NPAD // 16
E = 800000
LG = 128
NC = 2
NS = 16
NW = NC * NS
GPW = 200
GROUPS = NW * GPW
EPAD = GROUPS * LG
SDEPTH = 8

_f32 = jnp.float32
_i32 = jnp.int32


def _zero_buf(buf):
    def body(i, _):
        buf[pl.ds(i * 16, 16)] = jnp.zeros((16,), _f32)
        return _
    lax.fori_loop(0, SL // 16, body, None)


def _mesh():
    return plsc.VectorSubcoreMesh(core_axis_name="c", subcore_axis_name="s")


def _edge_pipeline(src_v, dst_v, ew_v, tsh, acc, gb, vb, semg, sems):
    def gathers(j, p):
        for c in range(3):
            pltpu.async_copy(tsh[c].at[src_v.at[j]], gb[c][p], semg[p])

    def wait_gathers(j, p):
        for c in range(3):
            pltpu.make_async_copy(tsh[c].at[src_v.at[j]], gb[c][p],
                                  semg[p]).wait()

    def scatters(j, p):
        for c in range(3):
            pltpu.async_copy(vb[c][p], acc[c].at[dst_v.at[j]], sems[p],
                             add=True)

    def wait_scatters(j, p):
        for c in range(3):
            pltpu.make_async_copy(vb[c][p], acc[c].at[dst_v.at[j]],
                                  sems[p]).wait()

    for p in (0, 1):
        gathers(p, p)

    def pair(jj, _):
        for p in (0, 1):
            j = jj * 2 + p
            wait_gathers(j, p)

            @pl.when(jj > 0)
            def _():
                wait_scatters(j - 2, p)

            for c in range(3):
                for i in range(LG // 16):
                    sj = pl.ds(i * 16, 16)
                    vb[c][p][sj] = gb[c][p][sj] * ew_v[j, sj]
            scatters(j, p)

            @pl.when(j + 2 < GPW)
            def _():
                gathers(j + 2, p)
        return _
    lax.fori_loop(0, GPW // 2, pair, None)
    for p in (0, 1):
        wait_scatters(GPW - 2 + p, p)


def _hop_scratch():
    return [
        pltpu.VMEM((GPW, LG), _i32),
        pltpu.VMEM((GPW, LG), _i32),
        pltpu.VMEM((GPW, LG), _f32),
        pltpu.VMEM((SL,), _f32),
        pltpu.VMEM((SL,), _f32),
        pltpu.VMEM((SL,), _f32),
        [pltpu.VMEM((LG,), _f32)] * 6,
        [pltpu.VMEM((LG,), _f32)] * 6,
        pltpu.VMEM_SHARED((NPAD,), _f32),
        pltpu.VMEM_SHARED((NPAD,), _f32),
        pltpu.VMEM_SHARED((NPAD,), _f32),
        pltpu.VMEM_SHARED((NPAD,), _f32),
        pltpu.VMEM_SHARED((NPAD,), _f32),
        pltpu.VMEM_SHARED((NPAD,), _f32),
        pltpu.SemaphoreType.DMA,
        pltpu.SemaphoreType.DMA,
        pltpu.SemaphoreType.DMA,
        pltpu.SemaphoreType.DMA,
    ]


def _deg_call(srcg, ewg):
    @functools.partial(
        pl.kernel,
        out_type=jax.ShapeDtypeStruct((NC * NPAD,), _f32),
        mesh=_mesh(),
        scratch_types=[
            pltpu.VMEM((GPW, LG), _i32),
            pltpu.VMEM((GPW, LG), _f32),
            pltpu.VMEM((SL,), _f32),
            pltpu.VMEM_SHARED((NPAD,), _f32),
            pltpu.SemaphoreType.DMA,
        ],
    )
    def k(src_ref, ew_ref, out_ref, src_v, ew_v, zbuf_v, acc_sh, semd):
        cid = lax.axis_index("c")
        sid = lax.axis_index("s")
        wid = sid * NC + cid
        _zero_buf(zbuf_v)
        pltpu.sync_copy(zbuf_v, acc_sh.at[pl.ds(sid * SL, SL)])
        plsc.subcore_barrier()
        g0 = wid * GPW
        pltpu.sync_copy(src_ref.at[pl.ds(g0, GPW)], src_v)
        pltpu.sync_copy(ew_ref.at[pl.ds(g0, GPW)], ew_v)

        def grp(j, _):
            pltpu.async_copy(ew_v.at[j], acc_sh.at[src_v.at[j]], semd,
                             add=True)

            @pl.when(j >= SDEPTH)
            def _():
                pltpu.make_async_copy(ew_v.at[j], acc_sh.at[src_v.at[j]],
                                      semd).wait()
            return _
        lax.fori_loop(0, GPW, grp, None)
        for _ in range(SDEPTH):
            pltpu.make_async_copy(ew_v.at[0], acc_sh.at[src_v.at[0]],
                                  semd).wait()
        plsc.subcore_barrier()
        pltpu.sync_copy(acc_sh.at[pl.ds(sid * SL, SL)], zbuf_v)
        pltpu.sync_copy(zbuf_v, out_ref.at[pl.ds(cid * NPAD + sid * SL, SL)])

    return k(srcg, ewg)


def _dinv_call(degp):
    def body(deg_ref, out_ref):
        d = deg_ref[0:1, :] + deg_ref[1:2, :]
        out_ref[...] = jnp.where(d > 0.0, lax.rsqrt(d), 0.0)

    return pl.pallas_call(
        body,
        out_shape=jax.ShapeDtypeStruct((1, NPAD), _f32),
    )(degp)


def _hop1_call(srcg, dstg, ewg, dinv, xflat):
    @functools.partial(
        pl.kernel,
        out_type=jax.ShapeDtypeStruct((NC * 3 * NPAD,), _f32),
        mesh=_mesh(),
        scratch_types=_hop_scratch(),
    )
    def k(src_ref, dst_ref, ew_ref, dinv_ref, x_ref, s_out,
          src_v, dst_v, ew_v, dbuf_v, zbuf_v, bbuf_v, gb6, vb6,
          t0_sh, t1_sh, t2_sh, a0_sh, a1_sh, a2_sh,
          semg0, semg1, sems0, sems1):
        tsh = (t0_sh, t1_sh, t2_sh)
        acc = (a0_sh, a1_sh, a2_sh)
        gb = (gb6[0:2], gb6[2:4], gb6[4:6])
        vb = (vb6[0:2], vb6[2:4], vb6[4:6])
        cid = lax.axis_index("c")
        sid = lax.axis_index("s")
        wid = sid * NC + cid
        sl = pl.ds(sid * SL, SL)

        pltpu.sync_copy(dinv_ref.at[sl], dbuf_v)
        for c in range(3):
            pltpu.sync_copy(x_ref.at[pl.ds(c * NPAD + sid * SL, SL)], zbuf_v)

            def mull(i, _):
                sj = pl.ds(i * 16, 16)
                zbuf_v[sj] = zbuf_v[sj] * dbuf_v[sj]
                return _
            lax.fori_loop(0, SL // 16, mull, None)
            pltpu.sync_copy(zbuf_v, tsh[c].at[sl])
        _zero_buf(zbuf_v)
        for c in range(3):
            pltpu.sync_copy(zbuf_v, acc[c].at[sl])
        plsc.subcore_barrier()

        g0 = wid * GPW
        pltpu.sync_copy(src_ref.at[pl.ds(g0, GPW)], src_v)
        pltpu.sync_copy(dst_ref.at[pl.ds(g0, GPW)], dst_v)
        pltpu.sync_copy(ew_ref.at[pl.ds(g0, GPW)], ew_v)
        _edge_pipeline(src_v, dst_v, ew_v, tsh, acc, gb, vb,
                       (semg0, semg1), (sems0, sems1))

        plsc.subcore_barrier()
        for c in range(3):
            pltpu.sync_copy(acc[c].at[sl], zbuf_v)
            pltpu.sync_copy(zbuf_v,
                            s_out.at[pl.ds((cid * 3 + c) * NPAD + sid * SL, SL)])

    return k(srcg, dstg, ewg, dinv, xflat)


def _hop2_call(srcg, dstg, ewg, dinv, s1p):
    @functools.partial(
        pl.kernel,
        out_type=jax.ShapeDtypeStruct((NC * 3 * NPAD,), _f32),
        mesh=_mesh(),
        scratch_types=_hop_scratch(),
    )
    def k(src_ref, dst_ref, ew_ref, dinv_ref, s1_ref, s_out,
          src_v, dst_v, ew_v, dbuf_v, zbuf_v, bbuf_v, gb6, vb6,
          t0_sh, t1_sh, t2_sh, a0_sh, a1_sh, a2_sh,
          semg0, semg1, sems0, sems1):
        tsh = (t0_sh, t1_sh, t2_sh)
        acc = (a0_sh, a1_sh, a2_sh)
        gb = (gb6[0:2], gb6[2:4], gb6[4:6])
        vb = (vb6[0:2], vb6[2:4], vb6[4:6])
        cid = lax.axis_index("c")
        sid = lax.axis_index("s")
        wid = sid * NC + cid
        sl = pl.ds(sid * SL, SL)

        pltpu.sync_copy(dinv_ref.at[sl], dbuf_v)
        for c in range(3):
            pltpu.sync_copy(s1_ref.at[pl.ds(c * NPAD + sid * SL, SL)], zbuf_v)
            pltpu.sync_copy(s1_ref.at[pl.ds((3 + c) * NPAD + sid * SL, SL)],
                            bbuf_v)

            def mull(i, _):
                sj = pl.ds(i * 16, 16)
                d = dbuf_v[sj]
                zbuf_v[sj] = -(zbuf_v[sj] + bbuf_v[sj]) * d * d
                return _
            lax.fori_loop(0, SL // 16, mull, None)
            pltpu.sync_copy(zbuf_v, tsh[c].at[sl])
        _zero_buf(zbuf_v)
        for c in range(3):
            pltpu.sync_copy(zbuf_v, acc[c].at[sl])
        plsc.subcore_barrier()

        g0 = wid * GPW
        pltpu.sync_copy(src_ref.at[pl.ds(g0, GPW)], src_v)
        pltpu.sync_copy(dst_ref.at[pl.ds(g0, GPW)], dst_v)
        pltpu.sync_copy(ew_ref.at[pl.ds(g0, GPW)], ew_v)
        _edge_pipeline(src_v, dst_v, ew_v, tsh, acc, gb, vb,
                       (semg0, semg1), (sems0, sems1))

        plsc.subcore_barrier()
        for c in range(3):
            pltpu.sync_copy(acc[c].at[sl], zbuf_v)
            pltpu.sync_copy(zbuf_v,
                            s_out.at[pl.ds((cid * 3 + c) * NPAD + sid * SL, SL)])

    return k(srcg, dstg, ewg, dinv, s1p)


_NB = 8
_BL = NPAD // _NB


def _final_call(xpad, s1p, s2p, dinv2, W9, gamma_c, beta_c):
    def body(x_ref, s1_ref, s2_ref, d_ref, w_ref, g_ref, b_ref, out_ref,
             acc_ref, st_ref):
        p = pl.program_id(0)
        j = pl.program_id(1)

        @pl.when(jnp.logical_and(p == 0, j == 0))
        def _():
            acc_ref[...] = jnp.zeros_like(acc_ref)

        nd = -d_ref[...]
        t0 = x_ref[...]
        t1 = nd * (s1_ref[0:3, :] + s1_ref[3:6, :])
        t2 = 2.0 * nd * (s2_ref[0:3, :] + s2_ref[3:6, :]) - t0
        t9 = jnp.concatenate([t0, t1, t2], axis=0)
        out64 = lax.dot_general(w_ref[...], t9, (((0,), (0,)), ((), ())),
                                preferred_element_type=_f32)

        @pl.when(p == 0)
        def _():
            acc_ref[:, 0:1] += jnp.sum(out64, axis=1, keepdims=True)
            acc_ref[:, 1:2] += jnp.sum(out64 * out64, axis=1, keepdims=True)
            out_ref[...] = out64

        @pl.when(p == 1)
        def _():
            @pl.when(j == 0)
            def _():
                mean = acc_ref[:, 0:1] * (1.0 / N)
                var = acc_ref[:, 1:2] * (1.0 / N) - mean * mean
                st_ref[:, 0:1] = mean
                st_ref[:, 1:2] = lax.rsqrt(var + 1e-5)
            mean = st_ref[:, 0:1]
            rstd = st_ref[:, 1:2]
            z = (out64 - mean) * rstd * g_ref[...] + b_ref[...]
            out_ref[...] = jnp.maximum(z, 0.0)

    return pl.pallas_call(
        body,
        grid=(2, _NB),
        in_specs=[
            pl.BlockSpec((3, _BL), lambda p, j: (0, j)),
            pl.BlockSpec((6, _BL), lambda p, j: (0, j)),
            pl.BlockSpec((6, _BL), lambda p, j: (0, j)),
            pl.BlockSpec((1, _BL), lambda p, j: (0, j)),
            pl.BlockSpec((9, 64), lambda p, j: (0, 0)),
            pl.BlockSpec((64, 1), lambda p, j: (0, 0)),
            pl.BlockSpec((64, 1), lambda p, j: (0, 0)),
        ],
        out_specs=pl.BlockSpec((64, _BL), lambda p, j: (0, j)),
        out_shape=jax.ShapeDtypeStruct((64, NPAD), _f32),
        scratch_shapes=[
            pltpu.VMEM((64, 2), _f32),
            pltpu.VMEM((64, 2), _f32),
        ],
    )(xpad, s1p, s2p, dinv2, W9, gamma_c, beta_c)


def kernel(x, edge_index, edge_weight, W, b, gamma, beta):
    del b
    xpad = jnp.pad(x[0], ((0, 0), (0, NPAD - N)))
    npad_e = EPAD - E
    pad_idx = (N + (jnp.arange(npad_e, dtype=_i32) % (NPAD - N)))
    src = jnp.concatenate([edge_index[0], pad_idx]).reshape(GROUPS, LG)
    dst = jnp.concatenate([edge_index[1], pad_idx]).reshape(GROUPS, LG)
    ew = jnp.concatenate([edge_weight,
                          jnp.zeros((npad_e,), _f32)]).reshape(GROUPS, LG)

    degp = _deg_call(src, ew).reshape(NC, NPAD)
    dinv2 = _dinv_call(degp)
    dinv = dinv2.reshape(NPAD)
    s1p = _hop1_call(src, dst, ew, dinv, xpad.reshape(-1))
    s2p = _hop2_call(src, dst, ew, dinv, s1p)

    W9 = W.reshape(9, 64)
    outT = _final_call(xpad, s1p.reshape(6, NPAD), s2p.reshape(6, NPAD),
                       dinv2, W9,
                       gamma.reshape(64, 1), beta.reshape(64, 1))
    return outT[:, :N].reshape(1, 64, N)


# speedup vs baseline: 78.1228x; 1.0002x over previous
import functools

import jax
import jax.numpy as jnp
from jax import lax
from jax.experimental import pallas as pl
from jax.experimental.pallas import tpu as pltpu
from jax.experimental.pallas import tpu_sc as plsc

N = 50000
NPAD = 50176
SL = NPAD // 16
E = 800000
LG = 128
NC = 2
NS = 16
NW = NC * NS
GPW = 200
GROUPS = NW * GPW
EPAD = GROUPS * LG
SDEPTH = 8

_f32 = jnp.float32
_i32 = jnp.int32


def _zero_buf(buf):
    def body(i, _):
        buf[pl.ds(i * 16, 16)] = jnp.zeros((16,), _f32)
        return _
    lax.fori_loop(0, SL // 16, body, None)


def _mesh():
    return plsc.VectorSubcoreMesh(core_axis_name="c", subcore_axis_name="s")


SB = 25
SBW = SB * LG
NSB = GPW // SB


def _edge_pipeline(g0, src_ref, dst_ref, ew_ref, tsh, acc,
                   srcb, dstb, ewb, gb, vb, seml, semg, sems):
    def eload(q, p):
        o = pl.ds(g0 + q * SBW, SBW)
        pltpu.async_copy(src_ref.at[o], srcb[p], seml[p])
        pltpu.async_copy(dst_ref.at[o], dstb[q % 4], seml[p])
        pltpu.async_copy(ew_ref.at[o], ewb[p], seml[p])

    def wait_eload(q, p):
        o = pl.ds(g0 + q * SBW, SBW)
        pltpu.make_async_copy(src_ref.at[o], srcb[p], seml[p]).wait()
        pltpu.make_async_copy(dst_ref.at[o], dstb[q % 4], seml[p]).wait()
        pltpu.make_async_copy(ew_ref.at[o], ewb[p], seml[p]).wait()

    def gathers(q, p):
        for c in range(3):
            pltpu.async_copy(tsh[c].at[srcb[p]], gb[c][p], semg[p])

    def wait_gathers(q, p):
        for c in range(3):
            pltpu.make_async_copy(tsh[c].at[srcb[p]], gb[c][p],
                                  semg[p]).wait()

    def scatters(q, p):
        for c in range(3):
            pltpu.async_copy(vb[c][p], acc[c].at[dstb[q % 4]], sems[p],
                             add=True)

    def wait_scatters(q, p):
        for c in range(3):
            pltpu.make_async_copy(vb[c][p], acc[c].at[dstb[q % 4]],
                                  sems[p]).wait()

    eload(0, 0)
    eload(1, 1)
    wait_eload(0, 0)
    gathers(0, 0)

    for q in range(NSB):
        p = q % 2
        wait_gathers(q, p)
        if q >= 2:
            wait_scatters(q - 2, p)

        def vec(i, _2, p=p):
            sj = pl.ds(i * 16, 16)
            for c in range(3):
                vb[c][p][sj] = gb[c][p][sj] * ewb[p][sj]
            return _2
        lax.fori_loop(0, SBW // 16, vec, None)
        scatters(q, p)
        if q + 1 < NSB:
            wait_eload(q + 1, 1 - p)
            gathers(q + 1, 1 - p)
        if q + 2 < NSB:
            eload(q + 2, p)
    for q in (NSB - 2, NSB - 1):
        wait_scatters(q, q % 2)


def _hop_scratch():
    return [
        [pltpu.VMEM((SBW,), _i32)] * 2,
        [pltpu.VMEM((SBW,), _i32)] * 4,
        [pltpu.VMEM((SBW,), _f32)] * 2,
        pltpu.VMEM((SL,), _f32),
        pltpu.VMEM((SL,), _f32),
        pltpu.VMEM((SL,), _f32),
        [pltpu.VMEM((SBW,), _f32)] * 6,
        [pltpu.VMEM((SBW,), _f32)] * 6,
        pltpu.VMEM_SHARED((NPAD,), _f32),
        pltpu.VMEM_SHARED((NPAD,), _f32),
        pltpu.VMEM_SHARED((NPAD,), _f32),
        pltpu.VMEM_SHARED((NPAD,), _f32),
        pltpu.VMEM_SHARED((NPAD,), _f32),
        pltpu.VMEM_SHARED((NPAD,), _f32),
        pltpu.SemaphoreType.DMA,
        pltpu.SemaphoreType.DMA,
        pltpu.SemaphoreType.DMA,
        pltpu.SemaphoreType.DMA,
        pltpu.SemaphoreType.DMA,
        pltpu.SemaphoreType.DMA,
    ]


def _deg_call(srcg, ewg):
    @functools.partial(
        pl.kernel,
        out_type=jax.ShapeDtypeStruct((NC * NPAD,), _f32),
        mesh=_mesh(),
        scratch_types=[
            pltpu.VMEM((GPW, LG), _i32),
            pltpu.VMEM((GPW, LG), _f32),
            pltpu.VMEM((SL,), _f32),
            pltpu.VMEM_SHARED((NPAD,), _f32),
            pltpu.SemaphoreType.DMA,
        ],
    )
    def k(src_ref, ew_ref, out_ref, src_v, ew_v, zbuf_v, acc_sh, semd):
        cid = lax.axis_index("c")
        sid = lax.axis_index("s")
        wid = sid * NC + cid
        _zero_buf(zbuf_v)
        pltpu.sync_copy(zbuf_v, acc_sh.at[pl.ds(sid * SL, SL)])
        plsc.subcore_barrier()
        g0 = wid * GPW
        pltpu.sync_copy(src_ref.at[pl.ds(g0, GPW)], src_v)
        pltpu.sync_copy(ew_ref.at[pl.ds(g0, GPW)], ew_v)

        def grp(j, _):
            pltpu.async_copy(ew_v.at[j], acc_sh.at[src_v.at[j]], semd,
                             add=True)

            @pl.when(j >= SDEPTH)
            def _():
                pltpu.make_async_copy(ew_v.at[j], acc_sh.at[src_v.at[j]],
                                      semd).wait()
            return _
        lax.fori_loop(0, GPW, grp, None)
        for _ in range(SDEPTH):
            pltpu.make_async_copy(ew_v.at[0], acc_sh.at[src_v.at[0]],
                                  semd).wait()
        plsc.subcore_barrier()
        pltpu.sync_copy(acc_sh.at[pl.ds(sid * SL, SL)], zbuf_v)
        pltpu.sync_copy(zbuf_v, out_ref.at[pl.ds(cid * NPAD + sid * SL, SL)])

    return k(srcg, ewg)


def _dinv_call(degp):
    def body(deg_ref, out_ref):
        d = deg_ref[0:1, :] + deg_ref[1:2, :]
        out_ref[...] = jnp.where(d > 0.0, lax.rsqrt(d), 0.0)

    return pl.pallas_call(
        body,
        out_shape=jax.ShapeDtypeStruct((1, NPAD), _f32),
    )(degp)


def _hop1_call(srcg, dstg, ewg, dinv, xflat):
    @functools.partial(
        pl.kernel,
        out_type=jax.ShapeDtypeStruct((NC * 3 * NPAD,), _f32),
        mesh=_mesh(),
        scratch_types=_hop_scratch(),
    )
    def k(src_ref, dst_ref, ew_ref, dinv_ref, x_ref, s_out,
          srcb, dstb, ewb, dbuf_v, zbuf_v, bbuf_v, gb6, vb6,
          t0_sh, t1_sh, t2_sh, a0_sh, a1_sh, a2_sh,
          seml0, seml1, semg0, semg1, sems0, sems1):
        tsh = (t0_sh, t1_sh, t2_sh)
        acc = (a0_sh, a1_sh, a2_sh)
        gb = (gb6[0:2], gb6[2:4], gb6[4:6])
        vb = (vb6[0:2], vb6[2:4], vb6[4:6])
        cid = lax.axis_index("c")
        sid = lax.axis_index("s")
        wid = sid * NC + cid
        sl = pl.ds(sid * SL, SL)

        pltpu.sync_copy(dinv_ref.at[sl], dbuf_v)
        for c in range(3):
            pltpu.sync_copy(x_ref.at[pl.ds(c * NPAD + sid * SL, SL)], zbuf_v)

            def mull(i, _):
                sj = pl.ds(i * 16, 16)
                zbuf_v[sj] = zbuf_v[sj] * dbuf_v[sj]
                return _
            lax.fori_loop(0, SL // 16, mull, None)
            pltpu.sync_copy(zbuf_v, tsh[c].at[sl])
        _zero_buf(zbuf_v)
        for c in range(3):
            pltpu.sync_copy(zbuf_v, acc[c].at[sl])
        plsc.subcore_barrier()

        _edge_pipeline(wid * NSB * SBW, src_ref, dst_ref, ew_ref, tsh, acc,
                       srcb, dstb, ewb, gb, vb, (seml0, seml1),
                       (semg0, semg1), (sems0, sems1))

        plsc.subcore_barrier()
        for c in range(3):
            pltpu.sync_copy(acc[c].at[sl], zbuf_v)
            pltpu.sync_copy(zbuf_v,
                            s_out.at[pl.ds((cid * 3 + c) * NPAD + sid * SL, SL)])

    return k(srcg, dstg, ewg, dinv, xflat)


def _hop2_call(srcg, dstg, ewg, dinv, s1p):
    @functools.partial(
        pl.kernel,
        out_type=jax.ShapeDtypeStruct((NC * 3 * NPAD,), _f32),
        mesh=_mesh(),
        scratch_types=_hop_scratch(),
    )
    def k(src_ref, dst_ref, ew_ref, dinv_ref, s1_ref, s_out,
          srcb, dstb, ewb, dbuf_v, zbuf_v, bbuf_v, gb6, vb6,
          t0_sh, t1_sh, t2_sh, a0_sh, a1_sh, a2_sh,
          seml0, seml1, semg0, semg1, sems0, sems1):
        tsh = (t0_sh, t1_sh, t2_sh)
        acc = (a0_sh, a1_sh, a2_sh)
        gb = (gb6[0:2], gb6[2:4], gb6[4:6])
        vb = (vb6[0:2], vb6[2:4], vb6[4:6])
        cid = lax.axis_index("c")
        sid = lax.axis_index("s")
        wid = sid * NC + cid
        sl = pl.ds(sid * SL, SL)

        pltpu.sync_copy(dinv_ref.at[sl], dbuf_v)
        for c in range(3):
            pltpu.sync_copy(s1_ref.at[pl.ds(c * NPAD + sid * SL, SL)], zbuf_v)
            pltpu.sync_copy(s1_ref.at[pl.ds((3 + c) * NPAD + sid * SL, SL)],
                            bbuf_v)

            def mull(i, _):
                sj = pl.ds(i * 16, 16)
                d = dbuf_v[sj]
                zbuf_v[sj] = -(zbuf_v[sj] + bbuf_v[sj]) * d * d
                return _
            lax.fori_loop(0, SL // 16, mull, None)
            pltpu.sync_copy(zbuf_v, tsh[c].at[sl])
        _zero_buf(zbuf_v)
        for c in range(3):
            pltpu.sync_copy(zbuf_v, acc[c].at[sl])
        plsc.subcore_barrier()

        _edge_pipeline(wid * NSB * SBW, src_ref, dst_ref, ew_ref, tsh, acc,
                       srcb, dstb, ewb, gb, vb, (seml0, seml1),
                       (semg0, semg1), (sems0, sems1))

        plsc.subcore_barrier()
        for c in range(3):
            pltpu.sync_copy(acc[c].at[sl], zbuf_v)
            pltpu.sync_copy(zbuf_v,
                            s_out.at[pl.ds((cid * 3 + c) * NPAD + sid * SL, SL)])

    return k(srcg, dstg, ewg, dinv, s1p)


_NB = 8
_BL = NPAD // _NB


def _final_call(xpad, s1p, s2p, dinv2, W9, gamma_c, beta_c):
    def body(x_ref, s1_ref, s2_ref, d_ref, w_ref, g_ref, b_ref, out_ref,
             acc_ref, st_ref):
        p = pl.program_id(0)
        j = pl.program_id(1)

        @pl.when(jnp.logical_and(p == 0, j == 0))
        def _():
            acc_ref[...] = jnp.zeros_like(acc_ref)

        nd = -d_ref[...]
        t0 = x_ref[...]
        t1 = nd * (s1_ref[0:3, :] + s1_ref[3:6, :])
        t2 = 2.0 * nd * (s2_ref[0:3, :] + s2_ref[3:6, :]) - t0
        t9 = jnp.concatenate([t0, t1, t2], axis=0)
        out64 = lax.dot_general(w_ref[...], t9, (((0,), (0,)), ((), ())),
                                preferred_element_type=_f32)

        @pl.when(p == 0)
        def _():
            acc_ref[:, 0:1] += jnp.sum(out64, axis=1, keepdims=True)
            acc_ref[:, 1:2] += jnp.sum(out64 * out64, axis=1, keepdims=True)
            out_ref[...] = out64

        @pl.when(p == 1)
        def _():
            @pl.when(j == 0)
            def _():
                mean = acc_ref[:, 0:1] * (1.0 / N)
                var = acc_ref[:, 1:2] * (1.0 / N) - mean * mean
                st_ref[:, 0:1] = mean
                st_ref[:, 1:2] = lax.rsqrt(var + 1e-5)
            mean = st_ref[:, 0:1]
            rstd = st_ref[:, 1:2]
            z = (out64 - mean) * rstd * g_ref[...] + b_ref[...]
            out_ref[...] = jnp.maximum(z, 0.0)

    return pl.pallas_call(
        body,
        grid=(2, _NB),
        in_specs=[
            pl.BlockSpec((3, _BL), lambda p, j: (0, j)),
            pl.BlockSpec((6, _BL), lambda p, j: (0, j)),
            pl.BlockSpec((6, _BL), lambda p, j: (0, j)),
            pl.BlockSpec((1, _BL), lambda p, j: (0, j)),
            pl.BlockSpec((9, 64), lambda p, j: (0, 0)),
            pl.BlockSpec((64, 1), lambda p, j: (0, 0)),
            pl.BlockSpec((64, 1), lambda p, j: (0, 0)),
        ],
        out_specs=pl.BlockSpec((64, _BL), lambda p, j: (0, j)),
        out_shape=jax.ShapeDtypeStruct((64, NPAD), _f32),
        scratch_shapes=[
            pltpu.VMEM((64, 2), _f32),
            pltpu.VMEM((64, 2), _f32),
        ],
    )(xpad, s1p, s2p, dinv2, W9, gamma_c, beta_c)


def kernel(x, edge_index, edge_weight, W, b, gamma, beta):
    del b
    xpad = jnp.pad(x[0], ((0, 0), (0, NPAD - N)))
    npad_e = EPAD - E
    pad_idx = (N + (jnp.arange(npad_e, dtype=_i32) % (NPAD - N)))
    src = jnp.concatenate([edge_index[0], pad_idx]).reshape(GROUPS, LG)
    dst = jnp.concatenate([edge_index[1], pad_idx]).reshape(GROUPS, LG)
    ew = jnp.concatenate([edge_weight,
                          jnp.zeros((npad_e,), _f32)]).reshape(GROUPS, LG)

    degp = _deg_call(src, ew).reshape(NC, NPAD)
    dinv2 = _dinv_call(degp)
    dinv = dinv2.reshape(NPAD)
    srcw = src.reshape(EPAD)
    dstw = dst.reshape(EPAD)
    eww = ew.reshape(EPAD)
    s1p = _hop1_call(srcw, dstw, eww, dinv, xpad.reshape(-1))
    s2p = _hop2_call(srcw, dstw, eww, dinv, s1p)

    W9 = W.reshape(9, 64)
    outT = _final_call(xpad, s1p.reshape(6, NPAD), s2p.reshape(6, NPAD),
                       dinv2, W9,
                       gamma.reshape(64, 1), beta.reshape(64, 1))
    return outT[:, :N].reshape(1, 64, N)


# speedup vs baseline: 80.2340x; 1.0270x over previous
import functools

import jax
import jax.numpy as jnp
from jax import lax
from jax.experimental import pallas as pl
from jax.experimental.pallas import tpu as pltpu
from jax.experimental.pallas import tpu_sc as plsc

N = 50000
NPAD = 50176
SL = NPAD // 16
E = 800000
LG = 128
NC = 2
NS = 16
NW = NC * NS
GPW = 200
GROUPS = NW * GPW
EPAD = GROUPS * LG
SDEPTH = 8

_f32 = jnp.float32
_i32 = jnp.int32


def _zero_buf(buf):
    def body(i, _):
        buf[pl.ds(i * 16, 16)] = jnp.zeros((16,), _f32)
        return _
    lax.fori_loop(0, SL // 16, body, None)


def _mesh():
    return plsc.VectorSubcoreMesh(core_axis_name="c", subcore_axis_name="s")


SB = 25
SBW = SB * LG
NSB = GPW // SB
EPW = NSB * SBW
NSB_REAL_LAST = (E - (NW - 1) * EPW) // SBW
PADE = NW * EPW - E


def _edge_pipeline(wid, src_ref, dst_ref, ew_ref, padi_ref, padw_ref,
                   tsh, acc, srcb, dstb, ewb, gb, vb, seml, semg, sems):
    g0 = wid * NSB * SBW

    def eload(q, p):
        o = pl.ds(g0 + q * SBW, SBW)
        if q < NSB_REAL_LAST:
            pltpu.async_copy(src_ref.at[o], srcb[p], seml[p])
            pltpu.async_copy(dst_ref.at[o], dstb[q % 4], seml[p])
            pltpu.async_copy(ew_ref.at[o], ewb[p], seml[p])
        else:
            op = pl.ds((q - NSB_REAL_LAST) * SBW, SBW)
            last = wid == NW - 1

            @pl.when(last)
            def _():
                pltpu.async_copy(padi_ref.at[op], srcb[p], seml[p])
                pltpu.async_copy(padi_ref.at[op], dstb[q % 4], seml[p])
                pltpu.async_copy(padw_ref.at[op], ewb[p], seml[p])

            @pl.when(jnp.logical_not(last))
            def _():
                pltpu.async_copy(src_ref.at[o], srcb[p], seml[p])
                pltpu.async_copy(dst_ref.at[o], dstb[q % 4], seml[p])
                pltpu.async_copy(ew_ref.at[o], ewb[p], seml[p])

    def wait_eload(q, p):
        pltpu.make_async_copy(padi_ref.at[pl.ds(0, SBW)], srcb[p],
                              seml[p]).wait()
        pltpu.make_async_copy(padi_ref.at[pl.ds(0, SBW)], dstb[q % 4],
                              seml[p]).wait()
        pltpu.make_async_copy(padw_ref.at[pl.ds(0, SBW)], ewb[p],
                              seml[p]).wait()

    def gathers(q, p):
        for c in range(3):
            pltpu.async_copy(tsh[c].at[srcb[p]], gb[c][p], semg[p])

    def wait_gathers(q, p):
        for c in range(3):
            pltpu.make_async_copy(tsh[c].at[srcb[p]], gb[c][p],
                                  semg[p]).wait()

    def scatters(q, p):
        for c in range(3):
            pltpu.async_copy(vb[c][p], acc[c].at[dstb[q % 4]], sems[p],
                             add=True)

    def wait_scatters(q, p):
        for c in range(3):
            pltpu.make_async_copy(vb[c][p], acc[c].at[dstb[q % 4]],
                                  sems[p]).wait()

    eload(0, 0)
    eload(1, 1)
    wait_eload(0, 0)
    gathers(0, 0)

    for q in range(NSB):
        p = q % 2
        wait_gathers(q, p)
        if q >= 2:
            wait_scatters(q - 2, p)

        def vec(i, _2, p=p):
            sj = pl.ds(i * 16, 16)
            for c in range(3):
                vb[c][p][sj] = gb[c][p][sj] * ewb[p][sj]
            return _2
        lax.fori_loop(0, SBW // 16, vec, None)
        scatters(q, p)
        if q + 1 < NSB:
            wait_eload(q + 1, 1 - p)
            gathers(q + 1, 1 - p)
        if q + 2 < NSB:
            eload(q + 2, p)
    for q in (NSB - 2, NSB - 1):
        wait_scatters(q, q % 2)


def _hop_scratch():
    return [
        [pltpu.VMEM((SBW,), _i32)] * 2,
        [pltpu.VMEM((SBW,), _i32)] * 4,
        [pltpu.VMEM((SBW,), _f32)] * 2,
        pltpu.VMEM((SL,), _f32),
        pltpu.VMEM((SL,), _f32),
        pltpu.VMEM((SL,), _f32),
        [pltpu.VMEM((SBW,), _f32)] * 6,
        [pltpu.VMEM((SBW,), _f32)] * 6,
        pltpu.VMEM_SHARED((NPAD,), _f32),
        pltpu.VMEM_SHARED((NPAD,), _f32),
        pltpu.VMEM_SHARED((NPAD,), _f32),
        pltpu.VMEM_SHARED((NPAD,), _f32),
        pltpu.VMEM_SHARED((NPAD,), _f32),
        pltpu.VMEM_SHARED((NPAD,), _f32),
        pltpu.SemaphoreType.DMA,
        pltpu.SemaphoreType.DMA,
        pltpu.SemaphoreType.DMA,
        pltpu.SemaphoreType.DMA,
        pltpu.SemaphoreType.DMA,
        pltpu.SemaphoreType.DMA,
    ]


def _deg_call(src_in, ew_in, padi, padw):
    @functools.partial(
        pl.kernel,
        out_type=jax.ShapeDtypeStruct((NC * NPAD,), _f32),
        mesh=_mesh(),
        scratch_types=[
            [pltpu.VMEM((SBW,), _i32)] * 4,
            [pltpu.VMEM((SBW,), _f32)] * 4,
            pltpu.VMEM((SL,), _f32),
            pltpu.VMEM_SHARED((NPAD,), _f32),
            pltpu.SemaphoreType.DMA,
            pltpu.SemaphoreType.DMA,
            pltpu.SemaphoreType.DMA,
        ],
    )
    def k(src_ref, ew_ref, padi_ref, padw_ref, out_ref,
          srcb, ewb, zbuf_v, acc_sh, seml0, seml1, semd):
        seml = (seml0, seml1)
        cid = lax.axis_index("c")
        sid = lax.axis_index("s")
        wid = sid * NC + cid
        g0 = wid * NSB * SBW
        _zero_buf(zbuf_v)
        pltpu.sync_copy(zbuf_v, acc_sh.at[pl.ds(sid * SL, SL)])
        plsc.subcore_barrier()

        def eload(q, p):
            o = pl.ds(g0 + q * SBW, SBW)
            if q < NSB_REAL_LAST:
                pltpu.async_copy(src_ref.at[o], srcb[q % 4], seml[p])
                pltpu.async_copy(ew_ref.at[o], ewb[q % 4], seml[p])
            else:
                op = pl.ds((q - NSB_REAL_LAST) * SBW, SBW)
                last = wid == NW - 1

                @pl.when(last)
                def _():
                    pltpu.async_copy(padi_ref.at[op], srcb[q % 4], seml[p])
                    pltpu.async_copy(padw_ref.at[op], ewb[q % 4], seml[p])

                @pl.when(jnp.logical_not(last))
                def _():
                    pltpu.async_copy(src_ref.at[o], srcb[q % 4], seml[p])
                    pltpu.async_copy(ew_ref.at[o], ewb[q % 4], seml[p])

        def wait_eload(q, p):
            pltpu.make_async_copy(padi_ref.at[pl.ds(0, SBW)], srcb[q % 4],
                                  seml[p]).wait()
            pltpu.make_async_copy(padw_ref.at[pl.ds(0, SBW)], ewb[q % 4],
                                  seml[p]).wait()

        def scat_desc(q):
            return pltpu.make_async_copy(ewb[q % 4], acc_sh.at[srcb[q % 4]],
                                         semd)

        eload(0, 0)
        eload(1, 1)
        for q in range(NSB):
            p = q % 2
            wait_eload(q, p)
            pltpu.async_copy(ewb[q % 4], acc_sh.at[srcb[q % 4]], semd,
                             add=True)
            if q >= 2:
                scat_desc(q - 2).wait()
            if q + 2 < NSB:
                eload(q + 2, p)
        for q in (NSB - 2, NSB - 1):
            scat_desc(q).wait()
        plsc.subcore_barrier()
        pltpu.sync_copy(acc_sh.at[pl.ds(sid * SL, SL)], zbuf_v)
        pltpu.sync_copy(zbuf_v, out_ref.at[pl.ds(cid * NPAD + sid * SL, SL)])

    return k(src_in, ew_in, padi, padw)


def _dinv_call(degp):
    def body(deg_ref, out_ref):
        d = deg_ref[0:1, :] + deg_ref[1:2, :]
        out_ref[...] = jnp.where(d > 0.0, lax.rsqrt(d), 0.0)

    return pl.pallas_call(
        body,
        out_shape=jax.ShapeDtypeStruct((1, NPAD), _f32),
    )(degp)


def _hop1_call(srcg, dstg, ewg, padi, padw, dinv, xflat):
    @functools.partial(
        pl.kernel,
        out_type=jax.ShapeDtypeStruct((NC * 3 * NPAD,), _f32),
        mesh=_mesh(),
        scratch_types=_hop_scratch(),
    )
    def k(src_ref, dst_ref, ew_ref, padi_ref, padw_ref, dinv_ref, x_ref,
          s_out,
          srcb, dstb, ewb, dbuf_v, zbuf_v, bbuf_v, gb6, vb6,
          t0_sh, t1_sh, t2_sh, a0_sh, a1_sh, a2_sh,
          seml0, seml1, semg0, semg1, sems0, sems1):
        tsh = (t0_sh, t1_sh, t2_sh)
        acc = (a0_sh, a1_sh, a2_sh)
        gb = (gb6[0:2], gb6[2:4], gb6[4:6])
        vb = (vb6[0:2], vb6[2:4], vb6[4:6])
        cid = lax.axis_index("c")
        sid = lax.axis_index("s")
        wid = sid * NC + cid
        sl = pl.ds(sid * SL, SL)

        pltpu.sync_copy(dinv_ref.at[sl], dbuf_v)
        for c in range(3):
            pltpu.sync_copy(x_ref.at[pl.ds(c * NPAD + sid * SL, SL)], zbuf_v)

            def mull(i, _):
                sj = pl.ds(i * 16, 16)
                zbuf_v[sj] = zbuf_v[sj] * dbuf_v[sj]
                return _
            lax.fori_loop(0, SL // 16, mull, None)
            pltpu.sync_copy(zbuf_v, tsh[c].at[sl])
        _zero_buf(zbuf_v)
        for c in range(3):
            pltpu.sync_copy(zbuf_v, acc[c].at[sl])
        plsc.subcore_barrier()

        _edge_pipeline(wid, src_ref, dst_ref, ew_ref, padi_ref, padw_ref,
                       tsh, acc, srcb, dstb, ewb, gb, vb, (seml0, seml1),
                       (semg0, semg1), (sems0, sems1))

        plsc.subcore_barrier()
        for c in range(3):
            pltpu.sync_copy(acc[c].at[sl], zbuf_v)
            pltpu.sync_copy(zbuf_v,
                            s_out.at[pl.ds((cid * 3 + c) * NPAD + sid * SL, SL)])

    return k(srcg, dstg, ewg, padi, padw, dinv, xflat)


def _hop2_call(srcg, dstg, ewg, padi, padw, dinv, s1p):
    @functools.partial(
        pl.kernel,
        out_type=jax.ShapeDtypeStruct((NC * 3 * NPAD,), _f32),
        mesh=_mesh(),
        scratch_types=_hop_scratch(),
    )
    def k(src_ref, dst_ref, ew_ref, padi_ref, padw_ref, dinv_ref, s1_ref,
          s_out,
          srcb, dstb, ewb, dbuf_v, zbuf_v, bbuf_v, gb6, vb6,
          t0_sh, t1_sh, t2_sh, a0_sh, a1_sh, a2_sh,
          seml0, seml1, semg0, semg1, sems0, sems1):
        tsh = (t0_sh, t1_sh, t2_sh)
        acc = (a0_sh, a1_sh, a2_sh)
        gb = (gb6[0:2], gb6[2:4], gb6[4:6])
        vb = (vb6[0:2], vb6[2:4], vb6[4:6])
        cid = lax.axis_index("c")
        sid = lax.axis_index("s")
        wid = sid * NC + cid
        sl = pl.ds(sid * SL, SL)

        pltpu.sync_copy(dinv_ref.at[sl], dbuf_v)
        for c in range(3):
            pltpu.sync_copy(s1_ref.at[pl.ds(c * NPAD + sid * SL, SL)], zbuf_v)
            pltpu.sync_copy(s1_ref.at[pl.ds((3 + c) * NPAD + sid * SL, SL)],
                            bbuf_v)

            def mull(i, _):
                sj = pl.ds(i * 16, 16)
                d = dbuf_v[sj]
                zbuf_v[sj] = -(zbuf_v[sj] + bbuf_v[sj]) * d * d
                return _
            lax.fori_loop(0, SL // 16, mull, None)
            pltpu.sync_copy(zbuf_v, tsh[c].at[sl])
        _zero_buf(zbuf_v)
        for c in range(3):
            pltpu.sync_copy(zbuf_v, acc[c].at[sl])
        plsc.subcore_barrier()

        _edge_pipeline(wid, src_ref, dst_ref, ew_ref, padi_ref, padw_ref,
                       tsh, acc, srcb, dstb, ewb, gb, vb, (seml0, seml1),
                       (semg0, semg1), (sems0, sems1))

        plsc.subcore_barrier()
        for c in range(3):
            pltpu.sync_copy(acc[c].at[sl], zbuf_v)
            pltpu.sync_copy(zbuf_v,
                            s_out.at[pl.ds((cid * 3 + c) * NPAD + sid * SL, SL)])

    return k(srcg, dstg, ewg, padi, padw, dinv, s1p)


_NB = 8
_BL = NPAD // _NB


def _final_call(xpad, s1p, s2p, dinv2, W9, gamma_c, beta_c):
    def body(x_ref, s1_ref, s2_ref, d_ref, w_ref, g_ref, b_ref, out_ref,
             acc_ref, st_ref):
        p = pl.program_id(0)
        j = pl.program_id(1)

        @pl.when(jnp.logical_and(p == 0, j == 0))
        def _():
            acc_ref[...] = jnp.zeros_like(acc_ref)

        nd = -d_ref[...]
        t0 = x_ref[...]
        t1 = nd * (s1_ref[0:3, :] + s1_ref[3:6, :])
        t2 = 2.0 * nd * (s2_ref[0:3, :] + s2_ref[3:6, :]) - t0
        t9 = jnp.concatenate([t0, t1, t2], axis=0)
        out64 = lax.dot_general(w_ref[...], t9, (((0,), (0,)), ((), ())),
                                preferred_element_type=_f32)

        @pl.when(p == 0)
        def _():
            acc_ref[:, 0:1] += jnp.sum(out64, axis=1, keepdims=True)
            acc_ref[:, 1:2] += jnp.sum(out64 * out64, axis=1, keepdims=True)
            out_ref[...] = out64

        @pl.when(p == 1)
        def _():
            @pl.when(j == 0)
            def _():
                mean = acc_ref[:, 0:1] * (1.0 / N)
                var = acc_ref[:, 1:2] * (1.0 / N) - mean * mean
                st_ref[:, 0:1] = mean
                st_ref[:, 1:2] = lax.rsqrt(var + 1e-5)
            mean = st_ref[:, 0:1]
            rstd = st_ref[:, 1:2]
            z = (out64 - mean) * rstd * g_ref[...] + b_ref[...]
            out_ref[...] = jnp.maximum(z, 0.0)

    return pl.pallas_call(
        body,
        grid=(2, _NB),
        in_specs=[
            pl.BlockSpec((3, _BL), lambda p, j: (0, j)),
            pl.BlockSpec((6, _BL), lambda p, j: (0, j)),
            pl.BlockSpec((6, _BL), lambda p, j: (0, j)),
            pl.BlockSpec((1, _BL), lambda p, j: (0, j)),
            pl.BlockSpec((9, 64), lambda p, j: (0, 0)),
            pl.BlockSpec((64, 1), lambda p, j: (0, 0)),
            pl.BlockSpec((64, 1), lambda p, j: (0, 0)),
        ],
        out_specs=pl.BlockSpec((64, _BL), lambda p, j: (0, j)),
        out_shape=jax.ShapeDtypeStruct((64, NPAD), _f32),
        scratch_shapes=[
            pltpu.VMEM((64, 2), _f32),
            pltpu.VMEM((64, 2), _f32),
        ],
    )(xpad, s1p, s2p, dinv2, W9, gamma_c, beta_c)


def kernel(x, edge_index, edge_weight, W, b, gamma, beta):
    del b
    xpad = jnp.pad(x[0], ((0, 0), (0, NPAD - N)))
    pad_idx = (N + (jnp.arange(PADE, dtype=_i32) % (NPAD - N)))
    pad_w = jnp.zeros((PADE,), _f32)
    src = edge_index[0]
    dst = edge_index[1]

    degp = _deg_call(src, edge_weight, pad_idx,
                     pad_w).reshape(NC, NPAD)
    dinv2 = _dinv_call(degp)
    dinv = dinv2.reshape(NPAD)
    s1p = _hop1_call(src, dst, edge_weight, pad_idx, pad_w, dinv,
                     xpad.reshape(-1))
    s2p = _hop2_call(src, dst, edge_weight, pad_idx, pad_w, dinv,
                     s1p)

    W9 = W.reshape(9, 64)
    outT = _final_call(xpad, s1p.reshape(6, NPAD), s2p.reshape(6, NPAD),
                       dinv2, W9,
                       gamma.reshape(64, 1), beta.reshape(64, 1))
    return outT[:, :N].reshape(1, 64, N)


# speedup vs baseline: 82.0700x; 1.0229x over previous
import functools

import jax
import jax.numpy as jnp
from jax import lax
from jax.experimental import pallas as pl
from jax.experimental.pallas import tpu as pltpu
from jax.experimental.pallas import tpu_sc as plsc

N = 50000
NPAD = 50176
SL = NPAD // 16
E = 800000
LG = 128
NC = 2
NS = 16
NW = NC * NS
GPW = 200
GROUPS = NW * GPW
EPAD = GROUPS * LG
SDEPTH = 8

_f32 = jnp.float32
_i32 = jnp.int32


def _zero_buf(buf):
    def body(i, _):
        buf[pl.ds(i * 16, 16)] = jnp.zeros((16,), _f32)
        return _
    lax.fori_loop(0, SL // 16, body, None)


def _mesh():
    return plsc.VectorSubcoreMesh(core_axis_name="c", subcore_axis_name="s")


SB = 25
SBW = SB * LG
NSB = GPW // SB
EPW = NSB * SBW
NSB_REAL_LAST = (E - (NW - 1) * EPW) // SBW
PADE = NW * EPW - E


def _edge_pipeline(wid, src_ref, dst_ref, ew_ref, padi_ref, padw_ref,
                   tsh, acc, srcb, dstb, ewb, gb, vb, seml, semg, sems,
                   mbuf=None):
    g0 = wid * NSB * SBW

    def eload(q, p):
        o = pl.ds(g0 + q * SBW, SBW)
        if q < NSB_REAL_LAST:
            pltpu.async_copy(src_ref.at[o], srcb[p], seml[p])
            pltpu.async_copy(dst_ref.at[o], dstb[q % 4], seml[p])
            pltpu.async_copy(ew_ref.at[o], ewb[p], seml[p])
        else:
            op = pl.ds((q - NSB_REAL_LAST) * SBW, SBW)
            last = wid == NW - 1

            @pl.when(last)
            def _():
                pltpu.async_copy(padi_ref.at[op], srcb[p], seml[p])
                pltpu.async_copy(padi_ref.at[op], dstb[q % 4], seml[p])
                pltpu.async_copy(padw_ref.at[op], ewb[p], seml[p])

            @pl.when(jnp.logical_not(last))
            def _():
                pltpu.async_copy(src_ref.at[o], srcb[p], seml[p])
                pltpu.async_copy(dst_ref.at[o], dstb[q % 4], seml[p])
                pltpu.async_copy(ew_ref.at[o], ewb[p], seml[p])

    def wait_eload(q, p):
        pltpu.make_async_copy(padi_ref.at[pl.ds(0, SBW)], srcb[p],
                              seml[p]).wait()
        pltpu.make_async_copy(padi_ref.at[pl.ds(0, SBW)], dstb[q % 4],
                              seml[p]).wait()
        pltpu.make_async_copy(padw_ref.at[pl.ds(0, SBW)], ewb[p],
                              seml[p]).wait()

    def gathers(q, p):
        for c in range(len(tsh)):
            pltpu.async_copy(tsh[c].at[srcb[p]], gb[c][p], semg[p])

    def wait_gathers(q, p):
        for c in range(len(tsh)):
            pltpu.make_async_copy(tsh[c].at[srcb[p]], gb[c][p],
                                  semg[p]).wait()

    def scatters(q, p):
        for c in range(3):
            pltpu.async_copy(vb[c][p], acc[c].at[dstb[q % 4]], sems[p],
                             add=True)

    def wait_scatters(q, p):
        for c in range(3):
            pltpu.make_async_copy(vb[c][p], acc[c].at[dstb[q % 4]],
                                  sems[p]).wait()

    eload(0, 0)
    eload(1, 1)
    wait_eload(0, 0)
    gathers(0, 0)

    for q in range(NSB):
        p = q % 2
        wait_gathers(q, p)
        if q >= 2:
            wait_scatters(q - 2, p)

        if mbuf is None:
            def vec(i, _2, p=p):
                sj = pl.ds(i * 16, 16)
                for c in range(3):
                    vb[c][p][sj] = gb[c][p][sj] * ewb[p][sj]
                return _2
        else:
            mbuf_f = mbuf.bitcast(_f32)

            def vec(i, _2, p=p):
                sj = pl.ds(i * 16, 16)
                ewv = ewb[p][sj]
                w = gb[0][p][sj]
                mbuf[0, :] = w & jnp.int32(-65536)
                mbuf[1, :] = w << 16
                vb[0][p][sj] = mbuf_f[0, :] * ewv
                vb[1][p][sj] = mbuf_f[1, :] * ewv
                vb[2][p][sj] = gb[1][p][sj] * ewv
                return _2
        lax.fori_loop(0, SBW // 16, vec, None)
        scatters(q, p)
        if q + 1 < NSB:
            wait_eload(q + 1, 1 - p)
            gathers(q + 1, 1 - p)
        if q + 2 < NSB:
            eload(q + 2, p)
    for q in (NSB - 2, NSB - 1):
        wait_scatters(q, q % 2)


def _hop_scratch():
    return [
        [pltpu.VMEM((SBW,), _i32)] * 2,
        [pltpu.VMEM((SBW,), _i32)] * 4,
        [pltpu.VMEM((SBW,), _f32)] * 2,
        pltpu.VMEM((SL,), _f32),
        pltpu.VMEM((SL,), _f32),
        pltpu.VMEM((SL,), _f32),
        [pltpu.VMEM((SBW,), _f32)] * 6,
        [pltpu.VMEM((SBW,), _f32)] * 6,
        pltpu.VMEM_SHARED((NPAD,), _f32),
        pltpu.VMEM_SHARED((NPAD,), _f32),
        pltpu.VMEM_SHARED((NPAD,), _f32),
        pltpu.VMEM_SHARED((NPAD,), _f32),
        pltpu.VMEM_SHARED((NPAD,), _f32),
        pltpu.VMEM_SHARED((NPAD,), _f32),
        pltpu.SemaphoreType.DMA,
        pltpu.SemaphoreType.DMA,
        pltpu.SemaphoreType.DMA,
        pltpu.SemaphoreType.DMA,
        pltpu.SemaphoreType.DMA,
        pltpu.SemaphoreType.DMA,
    ]


def _deg_call(src_in, ew_in, padi, padw):
    @functools.partial(
        pl.kernel,
        out_type=jax.ShapeDtypeStruct((NC * NPAD,), _f32),
        mesh=_mesh(),
        scratch_types=[
            [pltpu.VMEM((SBW,), _i32)] * 4,
            [pltpu.VMEM((SBW,), _f32)] * 4,
            pltpu.VMEM((SL,), _f32),
            pltpu.VMEM_SHARED((NPAD,), _f32),
            pltpu.SemaphoreType.DMA,
            pltpu.SemaphoreType.DMA,
            pltpu.SemaphoreType.DMA,
        ],
    )
    def k(src_ref, ew_ref, padi_ref, padw_ref, out_ref,
          srcb, ewb, zbuf_v, acc_sh, seml0, seml1, semd):
        seml = (seml0, seml1)
        cid = lax.axis_index("c")
        sid = lax.axis_index("s")
        wid = sid * NC + cid
        g0 = wid * NSB * SBW
        _zero_buf(zbuf_v)
        pltpu.sync_copy(zbuf_v, acc_sh.at[pl.ds(sid * SL, SL)])
        plsc.subcore_barrier()

        def eload(q, p):
            o = pl.ds(g0 + q * SBW, SBW)
            if q < NSB_REAL_LAST:
                pltpu.async_copy(src_ref.at[o], srcb[q % 4], seml[p])
                pltpu.async_copy(ew_ref.at[o], ewb[q % 4], seml[p])
            else:
                op = pl.ds((q - NSB_REAL_LAST) * SBW, SBW)
                last = wid == NW - 1

                @pl.when(last)
                def _():
                    pltpu.async_copy(padi_ref.at[op], srcb[q % 4], seml[p])
                    pltpu.async_copy(padw_ref.at[op], ewb[q % 4], seml[p])

                @pl.when(jnp.logical_not(last))
                def _():
                    pltpu.async_copy(src_ref.at[o], srcb[q % 4], seml[p])
                    pltpu.async_copy(ew_ref.at[o], ewb[q % 4], seml[p])

        def wait_eload(q, p):
            pltpu.make_async_copy(padi_ref.at[pl.ds(0, SBW)], srcb[q % 4],
                                  seml[p]).wait()
            pltpu.make_async_copy(padw_ref.at[pl.ds(0, SBW)], ewb[q % 4],
                                  seml[p]).wait()

        def scat_desc(q):
            return pltpu.make_async_copy(ewb[q % 4], acc_sh.at[srcb[q % 4]],
                                         semd)

        eload(0, 0)
        eload(1, 1)
        for q in range(NSB):
            p = q % 2
            wait_eload(q, p)
            pltpu.async_copy(ewb[q % 4], acc_sh.at[srcb[q % 4]], semd,
                             add=True)
            if q >= 2:
                scat_desc(q - 2).wait()
            if q + 2 < NSB:
                eload(q + 2, p)
        for q in (NSB - 2, NSB - 1):
            scat_desc(q).wait()
        plsc.subcore_barrier()
        pltpu.sync_copy(acc_sh.at[pl.ds(sid * SL, SL)], zbuf_v)
        pltpu.sync_copy(zbuf_v, out_ref.at[pl.ds(cid * NPAD + sid * SL, SL)])

    return k(src_in, ew_in, padi, padw)


def _dinv_call(degp, xpad):
    def body(deg_ref, x_ref, dinv_ref, u01_ref, u2_ref):
        d = deg_ref[0:1, :] + deg_ref[1:2, :]
        dinv = jnp.where(d > 0.0, lax.rsqrt(d), 0.0)
        dinv_ref[...] = dinv
        u0 = dinv * x_ref[0:1, :]
        u1 = dinv * x_ref[1:2, :]
        u2_ref[...] = dinv * x_ref[2:3, :]
        b0 = lax.bitcast_convert_type(u0, _i32)
        b1 = lax.bitcast_convert_type(u1, _i32)
        hi = (b0 + 0x8000) & jnp.int32(-65536)
        lo = lax.shift_right_logical(b1 + 0x8000, 16) & jnp.int32(0xFFFF)
        u01_ref[...] = hi | lo

    return pl.pallas_call(
        body,
        out_shape=(
            jax.ShapeDtypeStruct((1, NPAD), _f32),
            jax.ShapeDtypeStruct((1, NPAD), _i32),
            jax.ShapeDtypeStruct((1, NPAD), _f32),
        ),
    )(degp, xpad)


def _hop1_call(srcg, dstg, ewg, padi, padw, u01, u2):
    @functools.partial(
        pl.kernel,
        out_type=jax.ShapeDtypeStruct((NC * 3 * NPAD,), _f32),
        mesh=_mesh(),
        scratch_types=[
            [pltpu.VMEM((SBW,), _i32)] * 2,
            [pltpu.VMEM((SBW,), _i32)] * 4,
            [pltpu.VMEM((SBW,), _f32)] * 2,
            pltpu.VMEM((SL,), _i32),
            pltpu.VMEM((SL,), _f32),
            [pltpu.VMEM((SBW,), _i32)] * 2,
            [pltpu.VMEM((SBW,), _f32)] * 2,
            [pltpu.VMEM((SBW,), _f32)] * 6,
            pltpu.VMEM((2, 16), _i32),
            pltpu.VMEM_SHARED((NPAD,), _i32),
            pltpu.VMEM_SHARED((NPAD,), _f32),
            pltpu.VMEM_SHARED((NPAD,), _f32),
            pltpu.VMEM_SHARED((NPAD,), _f32),
            pltpu.VMEM_SHARED((NPAD,), _f32),
            pltpu.SemaphoreType.DMA,
            pltpu.SemaphoreType.DMA,
            pltpu.SemaphoreType.DMA,
            pltpu.SemaphoreType.DMA,
            pltpu.SemaphoreType.DMA,
            pltpu.SemaphoreType.DMA,
        ],
    )
    def k(src_ref, dst_ref, ew_ref, padi_ref, padw_ref, u01_ref, u2_ref,
          s_out,
          srcb, dstb, ewb, ibuf_v, zbuf_v, gb01, gb2, vb6, mbuf,
          t01_sh, t2_sh, a0_sh, a1_sh, a2_sh,
          seml0, seml1, semg0, semg1, sems0, sems1):
        tsh = (t01_sh, t2_sh)
        acc = (a0_sh, a1_sh, a2_sh)
        gb = (gb01, gb2)
        vb = (vb6[0:2], vb6[2:4], vb6[4:6])
        cid = lax.axis_index("c")
        sid = lax.axis_index("s")
        wid = sid * NC + cid
        sl = pl.ds(sid * SL, SL)

        pltpu.sync_copy(u01_ref.at[sl], ibuf_v)
        pltpu.sync_copy(ibuf_v, t01_sh.at[sl])
        pltpu.sync_copy(u2_ref.at[sl], zbuf_v)
        pltpu.sync_copy(zbuf_v, t2_sh.at[sl])
        _zero_buf(zbuf_v)
        for c in range(3):
            pltpu.sync_copy(zbuf_v, acc[c].at[sl])
        plsc.subcore_barrier()

        _edge_pipeline(wid, src_ref, dst_ref, ew_ref, padi_ref, padw_ref,
                       tsh, acc, srcb, dstb, ewb, gb, vb, (seml0, seml1),
                       (semg0, semg1), (sems0, sems1), mbuf=mbuf)

        plsc.subcore_barrier()
        for c in range(3):
            pltpu.sync_copy(acc[c].at[sl], zbuf_v)
            pltpu.sync_copy(zbuf_v,
                            s_out.at[pl.ds((cid * 3 + c) * NPAD + sid * SL, SL)])

    return k(srcg, dstg, ewg, padi, padw, u01, u2)


def _hop2_call(srcg, dstg, ewg, padi, padw, dinv, s1p):
    @functools.partial(
        pl.kernel,
        out_type=jax.ShapeDtypeStruct((NC * 3 * NPAD,), _f32),
        mesh=_mesh(),
        scratch_types=_hop_scratch(),
    )
    def k(src_ref, dst_ref, ew_ref, padi_ref, padw_ref, dinv_ref, s1_ref,
          s_out,
          srcb, dstb, ewb, dbuf_v, zbuf_v, bbuf_v, gb6, vb6,
          t0_sh, t1_sh, t2_sh, a0_sh, a1_sh, a2_sh,
          seml0, seml1, semg0, semg1, sems0, sems1):
        tsh = (t0_sh, t1_sh, t2_sh)
        acc = (a0_sh, a1_sh, a2_sh)
        gb = (gb6[0:2], gb6[2:4], gb6[4:6])
        vb = (vb6[0:2], vb6[2:4], vb6[4:6])
        cid = lax.axis_index("c")
        sid = lax.axis_index("s")
        wid = sid * NC + cid
        sl = pl.ds(sid * SL, SL)

        pltpu.sync_copy(dinv_ref.at[sl], dbuf_v)
        for c in range(3):
            pltpu.sync_copy(s1_ref.at[pl.ds(c * NPAD + sid * SL, SL)], zbuf_v)
            pltpu.sync_copy(s1_ref.at[pl.ds((3 + c) * NPAD + sid * SL, SL)],
                            bbuf_v)

            def mull(i, _):
                sj = pl.ds(i * 16, 16)
                d = dbuf_v[sj]
                zbuf_v[sj] = -(zbuf_v[sj] + bbuf_v[sj]) * d * d
                return _
            lax.fori_loop(0, SL // 16, mull, None)
            pltpu.sync_copy(zbuf_v, tsh[c].at[sl])
        _zero_buf(zbuf_v)
        for c in range(3):
            pltpu.sync_copy(zbuf_v, acc[c].at[sl])
        plsc.subcore_barrier()

        _edge_pipeline(wid, src_ref, dst_ref, ew_ref, padi_ref, padw_ref,
                       tsh, acc, srcb, dstb, ewb, gb, vb, (seml0, seml1),
                       (semg0, semg1), (sems0, sems1))

        plsc.subcore_barrier()
        for c in range(3):
            pltpu.sync_copy(acc[c].at[sl], zbuf_v)
            pltpu.sync_copy(zbuf_v,
                            s_out.at[pl.ds((cid * 3 + c) * NPAD + sid * SL, SL)])

    return k(srcg, dstg, ewg, padi, padw, dinv, s1p)


_NB = 8
_BL = NPAD // _NB


def _final_call(xpad, s1p, s2p, dinv2, W9, gamma_c, beta_c):
    def body(x_ref, s1_ref, s2_ref, d_ref, w_ref, g_ref, b_ref, out_ref,
             acc_ref, st_ref):
        p = pl.program_id(0)
        j = pl.program_id(1)

        @pl.when(jnp.logical_and(p == 0, j == 0))
        def _():
            acc_ref[...] = jnp.zeros_like(acc_ref)

        nd = -d_ref[...]
        t0 = x_ref[...]
        t1 = nd * (s1_ref[0:3, :] + s1_ref[3:6, :])
        t2 = 2.0 * nd * (s2_ref[0:3, :] + s2_ref[3:6, :]) - t0
        t9 = jnp.concatenate([t0, t1, t2], axis=0)
        out64 = lax.dot_general(w_ref[...], t9, (((0,), (0,)), ((), ())),
                                preferred_element_type=_f32)

        @pl.when(p == 0)
        def _():
            acc_ref[:, 0:1] += jnp.sum(out64, axis=1, keepdims=True)
            acc_ref[:, 1:2] += jnp.sum(out64 * out64, axis=1, keepdims=True)
            out_ref[...] = out64

        @pl.when(p == 1)
        def _():
            @pl.when(j == 0)
            def _():
                mean = acc_ref[:, 0:1] * (1.0 / N)
                var = acc_ref[:, 1:2] * (1.0 / N) - mean * mean
                st_ref[:, 0:1] = mean
                st_ref[:, 1:2] = lax.rsqrt(var + 1e-5)
            mean = st_ref[:, 0:1]
            rstd = st_ref[:, 1:2]
            z = (out64 - mean) * rstd * g_ref[...] + b_ref[...]
            out_ref[...] = jnp.maximum(z, 0.0)

    return pl.pallas_call(
        body,
        grid=(2, _NB),
        in_specs=[
            pl.BlockSpec((3, _BL), lambda p, j: (0, j)),
            pl.BlockSpec((6, _BL), lambda p, j: (0, j)),
            pl.BlockSpec((6, _BL), lambda p, j: (0, j)),
            pl.BlockSpec((1, _BL), lambda p, j: (0, j)),
            pl.BlockSpec((9, 64), lambda p, j: (0, 0)),
            pl.BlockSpec((64, 1), lambda p, j: (0, 0)),
            pl.BlockSpec((64, 1), lambda p, j: (0, 0)),
        ],
        out_specs=pl.BlockSpec((64, _BL), lambda p, j: (0, j)),
        out_shape=jax.ShapeDtypeStruct((64, NPAD), _f32),
        scratch_shapes=[
            pltpu.VMEM((64, 2), _f32),
            pltpu.VMEM((64, 2), _f32),
        ],
    )(xpad, s1p, s2p, dinv2, W9, gamma_c, beta_c)


def kernel(x, edge_index, edge_weight, W, b, gamma, beta):
    del b
    xpad = jnp.pad(x[0], ((0, 0), (0, NPAD - N)))
    pad_idx = (N + (jnp.arange(PADE, dtype=_i32) % (NPAD - N)))
    pad_w = jnp.zeros((PADE,), _f32)
    src = edge_index[0]
    dst = edge_index[1]

    degp = _deg_call(src, edge_weight, pad_idx,
                     pad_w).reshape(NC, NPAD)
    dinv2, u01, u2 = _dinv_call(degp, xpad)
    dinv = dinv2.reshape(NPAD)
    s1p = _hop1_call(src, dst, edge_weight, pad_idx, pad_w,
                     u01.reshape(NPAD), u2.reshape(NPAD))
    s2p = _hop2_call(src, dst, edge_weight, pad_idx, pad_w, dinv,
                     s1p)

    W9 = W.reshape(9, 64)
    outT = _final_call(xpad, s1p.reshape(6, NPAD), s2p.reshape(6, NPAD),
                       dinv2, W9,
                       gamma.reshape(64, 1), beta.reshape(64, 1))
    return outT[:, :N].reshape(1, 64, N)
